# baseline (speedup 1.0000x reference)
import jax
import jax.numpy as jnp
from jax.experimental import pallas as pl
from jax.experimental.pallas import tpu as pltpu

_NNODES = 208
_NFEAT = 512
_NHID = 256
_NCLASS = 64
_FLAT = _NNODES * _NCLASS
_NSLICES = 8
_ROWS = 128 // _NSLICES
_DOTBLK = 8
_ROWBLK = 16
_RB = _ROWBLK * _NCLASS


def _fused(x_hbm, adj_hbm, w1_hbm, b1_ref, w2_ref, b2_ref,
           fc1w_hbm, fc1b_ref, fc2w_ref, fc2b_ref, fc3w_ref, fc3b_ref,
           out_ref, x_v, adj_v, w1_v, fc1_v, hflat_ref, sems, fsem):
    f32 = jnp.float32
    cp_x = pltpu.make_async_copy(x_hbm, x_v, fsem.at[0])
    cp_w1 = pltpu.make_async_copy(w1_hbm, w1_v, fsem.at[1])
    cp_adj = pltpu.make_async_copy(adj_hbm, adj_v, fsem.at[2])
    cp_x.start()
    cp_w1.start()
    cp_adj.start()
    fc1_cps = [
        pltpu.make_async_copy(
            fc1w_hbm.at[k * _ROWS:(k + 1) * _ROWS, :],
            fc1_v.at[k * _ROWS:(k + 1) * _ROWS, :], sems.at[k])
        for k in range(_NSLICES)]
    for cp in fc1_cps:
        cp.start()

    cp_x.wait()
    cp_w1.wait()
    s1 = jnp.dot(x_v[...], w1_v[...], preferred_element_type=f32)
    cp_adj.wait()
    adj = adj_v[...]
    h1 = jnp.maximum(jnp.dot(adj, s1, preferred_element_type=f32) + b1_ref[...], 0.0)
    s2 = jnp.dot(h1, w2_ref[...], preferred_element_type=f32)
    h2 = jnp.maximum(jnp.dot(adj, s2, preferred_element_type=f32) + b2_ref[...], 0.0)

    col = jax.lax.broadcasted_iota(jnp.int32, (_NCLASS, _RB), 1)
    tile = (col % _NCLASS == jax.lax.broadcasted_iota(
        jnp.int32, (_NCLASS, _RB), 0)).astype(f32)
    band = (jax.lax.broadcasted_iota(jnp.int32, (_ROWBLK, _RB), 1)
            // _NCLASS == jax.lax.broadcasted_iota(
                jnp.int32, (_ROWBLK, _RB), 0))
    zero = jnp.zeros((_ROWBLK, _RB), f32)
    for r in range(_NNODES // _ROWBLK):
        expand = jnp.dot(h2[r * _ROWBLK:(r + 1) * _ROWBLK, :], tile,
                         preferred_element_type=f32)
        hflat_ref[:, r * _RB:(r + 1) * _RB] = jnp.sum(
            jnp.where(band, expand, zero), axis=0, keepdims=True)
    hflat = hflat_ref[...]

    for k in range(6):
        fc1_cps[k].wait()
    p0 = jax.lax.dot_general(hflat, fc1_v[:6 * _ROWS, :],
                             (((1,), (1,)), ((), ())),
                             preferred_element_type=f32)
    for k in range(6, _NSLICES):
        fc1_cps[k].wait()
    p1 = jax.lax.dot_general(hflat, fc1_v[6 * _ROWS:, :],
                             (((1,), (1,)), ((), ())),
                             preferred_element_type=f32)
    f1 = jnp.concatenate([p0, p1], axis=1)
    f1 = jnp.maximum(f1 + fc1b_ref[...], 0.0)
    f2 = jnp.sum(fc2w_ref[...] * f1, axis=1, keepdims=True)
    f2 = jnp.maximum(f2 + fc2b_ref[...], 0.0)
    f3 = jnp.sum(f2 * fc3w_ref[...], keepdims=True) + fc3b_ref[...]
    out_ref[...] = jax.nn.sigmoid(f3)


def kernel(x, adj, W1, b1, W2, b2, fc1_w, fc1_b, fc2_w, fc2_b, fc3_w, fc3_b):
    hbm = pl.BlockSpec(memory_space=pltpu.MemorySpace.HBM)
    vmem = pl.BlockSpec(memory_space=pltpu.MemorySpace.VMEM)
    out = pl.pallas_call(
        _fused,
        in_specs=[hbm, hbm, hbm, vmem, vmem, vmem,
                  hbm, vmem, vmem, vmem, vmem, vmem],
        out_specs=vmem,
        out_shape=jax.ShapeDtypeStruct((1, 1), jnp.float32),
        scratch_shapes=[
            pltpu.VMEM((_NNODES, _NFEAT), jnp.float32),
            pltpu.VMEM((_NNODES, _NNODES), jnp.float32),
            pltpu.VMEM((_NFEAT, _NHID), jnp.float32),
            pltpu.VMEM((128, _FLAT), jnp.float32),
            pltpu.VMEM((1, _FLAT), jnp.float32),
            pltpu.SemaphoreType.DMA((_NSLICES,)),
            pltpu.SemaphoreType.DMA((3,)),
        ],
    )(x, adj, W1, b1.reshape(1, -1), W2, b2.reshape(1, -1),
      fc1_w, fc1_b.reshape(1, -1), fc2_w, fc2_b.reshape(-1, 1),
      fc3_w.reshape(-1, 1), fc3_b.reshape(1, 1))
    return out.reshape(1)


# speedup vs baseline: 1.0411x; 1.0411x over previous
import jax
import jax.numpy as jnp
from jax.experimental import pallas as pl
from jax.experimental.pallas import tpu as pltpu

_NNODES = 208
_NFEAT = 512
_NHID = 256
_NCLASS = 64
_FLAT = _NNODES * _NCLASS
_NSLICES = 8
_ROWS = 128 // _NSLICES
_DOTBLK = 8
_ROWBLK = 16
_RB = _ROWBLK * _NCLASS


def _fused(x_hbm, adj_hbm, w1_hbm, b1_ref, w2_ref, b2_ref,
           fc1w_hbm, fc1b_ref, fc2w_ref, fc2b_ref, fc3w_ref, fc3b_ref,
           out_ref, x_v, adj_v, w1_v, fc1_v, hflat_ref, sems, fsem):
    f32 = jnp.float32
    cp_x = pltpu.make_async_copy(x_hbm, x_v, fsem.at[0])
    cp_w1 = pltpu.make_async_copy(w1_hbm, w1_v, fsem.at[1])
    cp_adj = pltpu.make_async_copy(adj_hbm, adj_v, fsem.at[2])
    cp_x.start()
    cp_w1.start()
    cp_adj.start()
    fc1_cps = [
        pltpu.make_async_copy(
            fc1w_hbm.at[k * _ROWS:(k + 1) * _ROWS, :],
            fc1_v.at[k * _ROWS:(k + 1) * _ROWS, :], sems.at[k])
        for k in range(_NSLICES)]
    for cp in fc1_cps:
        cp.start()

    cp_x.wait()
    cp_w1.wait()
    s1 = jnp.dot(x_v[...], w1_v[...], preferred_element_type=f32)
    cp_adj.wait()
    adj = adj_v[...]
    h1 = jnp.maximum(jnp.dot(adj, s1, preferred_element_type=f32) + b1_ref[...], 0.0)
    s2 = jnp.dot(h1, w2_ref[...], preferred_element_type=f32)
    h2 = jnp.maximum(jnp.dot(adj, s2, preferred_element_type=f32) + b2_ref[...], 0.0)

    col = jax.lax.broadcasted_iota(jnp.int32, (_NCLASS, _RB), 1)
    tile = (col % _NCLASS == jax.lax.broadcasted_iota(
        jnp.int32, (_NCLASS, _RB), 0)).astype(f32)
    band = (jax.lax.broadcasted_iota(jnp.int32, (_ROWBLK, _RB), 1)
            // _NCLASS == jax.lax.broadcasted_iota(
                jnp.int32, (_ROWBLK, _RB), 0))
    zero = jnp.zeros((_ROWBLK, _RB), f32)
    for r in range(_NNODES // _ROWBLK):
        expand = jnp.dot(h2[r * _ROWBLK:(r + 1) * _ROWBLK, :], tile,
                         preferred_element_type=f32)
        hflat_ref[:, r * _RB:(r + 1) * _RB] = jnp.sum(
            jnp.where(band, expand, zero), axis=0, keepdims=True)
    hflat = hflat_ref[...]

    for cp in fc1_cps:
        cp.wait()
    f1 = jax.lax.dot_general(hflat, fc1_v[...], (((1,), (1,)), ((), ())),
                             preferred_element_type=f32)
    f1 = jnp.maximum(f1 + fc1b_ref[...], 0.0)
    f2 = jnp.sum(fc2w_ref[...] * f1, axis=1, keepdims=True)
    f2 = jnp.maximum(f2 + fc2b_ref[...], 0.0)
    f3 = jnp.sum(f2 * fc3w_ref[...], keepdims=True) + fc3b_ref[...]
    out_ref[...] = jax.nn.sigmoid(f3)


def kernel(x, adj, W1, b1, W2, b2, fc1_w, fc1_b, fc2_w, fc2_b, fc3_w, fc3_b):
    hbm = pl.BlockSpec(memory_space=pltpu.MemorySpace.HBM)
    vmem = pl.BlockSpec(memory_space=pltpu.MemorySpace.VMEM)
    out = pl.pallas_call(
        _fused,
        in_specs=[hbm, hbm, hbm, vmem, vmem, vmem,
                  hbm, vmem, vmem, vmem, vmem, vmem],
        out_specs=vmem,
        out_shape=jax.ShapeDtypeStruct((1, 1), jnp.float32),
        scratch_shapes=[
            pltpu.VMEM((_NNODES, _NFEAT), jnp.float32),
            pltpu.VMEM((_NNODES, _NNODES), jnp.float32),
            pltpu.VMEM((_NFEAT, _NHID), jnp.float32),
            pltpu.VMEM((128, _FLAT), jnp.float32),
            pltpu.VMEM((1, _FLAT), jnp.float32),
            pltpu.SemaphoreType.DMA((_NSLICES,)),
            pltpu.SemaphoreType.DMA((3,)),
        ],
    )(x, adj, W1, b1.reshape(1, -1), W2, b2.reshape(1, -1),
      fc1_w, fc1_b.reshape(1, -1), fc2_w, fc2_b.reshape(-1, 1),
      fc3_w.reshape(-1, 1), fc3_b.reshape(1, 1))
    return out.reshape(1)


# speedup vs baseline: 1.1096x; 1.0657x over previous
import jax
import jax.numpy as jnp
from jax.experimental import pallas as pl
from jax.experimental.pallas import tpu as pltpu

_NNODES = 208
_NFEAT = 512
_NHID = 256
_NCLASS = 64
_FLAT = _NNODES * _NCLASS
_NSLICES = 1
_ROWS = 128 // _NSLICES
_DOTBLK = 8
_ROWBLK = 16
_RB = _ROWBLK * _NCLASS


def _fused(x_hbm, adj_hbm, w1_hbm, b1_ref, w2_ref, b2_ref,
           fc1w_hbm, fc1b_ref, fc2w_ref, fc2b_ref, fc3w_ref, fc3b_ref,
           out_ref, x_v, adj_v, w1_v, fc1_v, hflat_ref, sems, fsem):
    f32 = jnp.float32
    cp_x = pltpu.make_async_copy(x_hbm, x_v, fsem.at[0])
    cp_w1 = pltpu.make_async_copy(w1_hbm, w1_v, fsem.at[1])
    cp_adj = pltpu.make_async_copy(adj_hbm, adj_v, fsem.at[2])
    cp_x.start()
    cp_w1.start()
    cp_adj.start()
    fc1_cps = [
        pltpu.make_async_copy(
            fc1w_hbm.at[k * _ROWS:(k + 1) * _ROWS, :],
            fc1_v.at[k * _ROWS:(k + 1) * _ROWS, :], sems.at[k])
        for k in range(_NSLICES)]
    for cp in fc1_cps:
        cp.start()

    cp_x.wait()
    cp_w1.wait()
    s1 = jnp.dot(x_v[...], w1_v[...], preferred_element_type=f32)
    cp_adj.wait()
    adj = adj_v[...]
    h1 = jnp.maximum(jnp.dot(adj, s1, preferred_element_type=f32) + b1_ref[...], 0.0)
    s2 = jnp.dot(h1, w2_ref[...], preferred_element_type=f32)
    h2 = jnp.maximum(jnp.dot(adj, s2, preferred_element_type=f32) + b2_ref[...], 0.0)

    col = jax.lax.broadcasted_iota(jnp.int32, (_NCLASS, _RB), 1)
    tile = (col % _NCLASS == jax.lax.broadcasted_iota(
        jnp.int32, (_NCLASS, _RB), 0)).astype(f32)
    band = (jax.lax.broadcasted_iota(jnp.int32, (_ROWBLK, _RB), 1)
            // _NCLASS == jax.lax.broadcasted_iota(
                jnp.int32, (_ROWBLK, _RB), 0))
    zero = jnp.zeros((_ROWBLK, _RB), f32)
    for r in range(_NNODES // _ROWBLK):
        expand = jnp.dot(h2[r * _ROWBLK:(r + 1) * _ROWBLK, :], tile,
                         preferred_element_type=f32)
        hflat_ref[:, r * _RB:(r + 1) * _RB] = jnp.sum(
            jnp.where(band, expand, zero), axis=0, keepdims=True)
    hflat = hflat_ref[...]

    for cp in fc1_cps:
        cp.wait()
    f1 = jax.lax.dot_general(hflat, fc1_v[...], (((1,), (1,)), ((), ())),
                             preferred_element_type=f32)
    f1 = jnp.maximum(f1 + fc1b_ref[...], 0.0)
    f2 = jnp.sum(fc2w_ref[...] * f1, axis=1, keepdims=True)
    f2 = jnp.maximum(f2 + fc2b_ref[...], 0.0)
    f3 = jnp.sum(f2 * fc3w_ref[...], keepdims=True) + fc3b_ref[...]
    out_ref[...] = jax.nn.sigmoid(f3)


def kernel(x, adj, W1, b1, W2, b2, fc1_w, fc1_b, fc2_w, fc2_b, fc3_w, fc3_b):
    hbm = pl.BlockSpec(memory_space=pltpu.MemorySpace.HBM)
    vmem = pl.BlockSpec(memory_space=pltpu.MemorySpace.VMEM)
    out = pl.pallas_call(
        _fused,
        in_specs=[hbm, hbm, hbm, vmem, vmem, vmem,
                  hbm, vmem, vmem, vmem, vmem, vmem],
        out_specs=vmem,
        out_shape=jax.ShapeDtypeStruct((1, 1), jnp.float32),
        scratch_shapes=[
            pltpu.VMEM((_NNODES, _NFEAT), jnp.float32),
            pltpu.VMEM((_NNODES, _NNODES), jnp.float32),
            pltpu.VMEM((_NFEAT, _NHID), jnp.float32),
            pltpu.VMEM((128, _FLAT), jnp.float32),
            pltpu.VMEM((1, _FLAT), jnp.float32),
            pltpu.SemaphoreType.DMA((_NSLICES,)),
            pltpu.SemaphoreType.DMA((3,)),
        ],
    )(x, adj, W1, b1.reshape(1, -1), W2, b2.reshape(1, -1),
      fc1_w, fc1_b.reshape(1, -1), fc2_w, fc2_b.reshape(-1, 1),
      fc3_w.reshape(-1, 1), fc3_b.reshape(1, 1))
    return out.reshape(1)
